# baseline (speedup 1.0000x reference)
import functools

import jax
import jax.numpy as jnp
from jax import lax
from jax.experimental import pallas as pl
from jax.experimental.pallas import tpu as pltpu
from jax.experimental.pallas import tpu_sc as plsc

N = 10000
NP = 10240
E = 320000
D_FEAT = 128
HIDDEN = 64
CH = 128
NW = 32
NCH_W = 80
EPAD = NW * NCH_W * CH
NCH_S = 160
ROWS_T = NP // 16


def _rsqrt_nr(x):
    i = jnp.int32(0x5F3759DF) - (lax.bitcast_convert_type(x, jnp.int32) >> 1)
    r = lax.bitcast_convert_type(i, jnp.float32)
    for _ in range(3):
        r = r * (1.5 - 0.5 * x * r * r)
    return r


def _sc_kernel(src_t, dst_t, ew_t, xw, acc_out, c2_out, deg_out,
               spm_acc, spm_sc, dinv_v, zbuf, srcb, dstb, ewb,
               rows_v, mrow4, crow4,
               gsem0, gsem1, gsem2, gsem3, ssem0, ssem1, ssem2, ssem3,
               csem, dsem, zsem):
    c = lax.axis_index("c")
    s = lax.axis_index("s")
    wid = s * 2 + c
    base = s * ROWS_T
    gsem = [gsem0, gsem1, gsem2, gsem3]
    ssem = [ssem0, ssem1, ssem2, ssem3]

    def zrow(r, _):
        for k4 in range(4):
            rows_v[r, pl.ds(k4 * 16, 16)] = jnp.zeros((16,), jnp.float32)
        return 0
    lax.fori_loop(0, CH, zrow, 0)

    def zflat(i, _):
        zbuf[pl.ds(i * 16, 16)] = jnp.zeros((16,), jnp.float32)
        return 0
    lax.fori_loop(0, ROWS_T // 16, zflat, 0)

    for k in range(ROWS_T // CH):
        pltpu.async_copy(rows_v.at[pl.ds(0, CH)], spm_acc.at[pl.ds(base + k * CH, CH)], zsem)
    pltpu.async_copy(zbuf, spm_sc.at[pl.ds(base, ROWS_T)], zsem)
    for k in range(ROWS_T // CH):
        pltpu.make_async_copy(rows_v.at[pl.ds(0, CH)], spm_acc.at[pl.ds(base + k * CH, CH)], zsem).wait()
    pltpu.make_async_copy(zbuf, spm_sc.at[pl.ds(base, ROWS_T)], zsem).wait()
    plsc.subcore_barrier()

    for half in range(2):
        row0 = s * NCH_S + half * NCH_W
        pltpu.sync_copy(dst_t.at[pl.ds(row0, NCH_W)], dstb)
        pltpu.sync_copy(ew_t.at[pl.ds(row0, NCH_W)], ewb)

        def deg_body(g, _):
            pltpu.async_copy(ewb.at[g], spm_sc.at[dstb.at[g]], dsem, add=True)
            return 0
        lax.fori_loop(0, NCH_W, deg_body, 0)

        def deg_drain(g, _):
            pltpu.make_async_copy(ewb.at[g], spm_sc.at[dstb.at[g]], dsem).wait()
            return 0
        lax.fori_loop(0, NCH_W, deg_drain, 0)
    plsc.subcore_barrier()

    pltpu.sync_copy(spm_sc, dinv_v)

    @pl.when(c == 0)
    def _():
        pltpu.sync_copy(spm_sc.at[pl.ds(base, ROWS_T)], deg_out.at[pl.ds(base, ROWS_T)])
    plsc.subcore_barrier()
    pltpu.async_copy(zbuf, spm_sc.at[pl.ds(base, ROWS_T)], zsem)

    row0 = wid * NCH_W
    pltpu.sync_copy(src_t.at[pl.ds(row0, NCH_W)], srcb)
    pltpu.sync_copy(dst_t.at[pl.ds(row0, NCH_W)], dstb)
    pltpu.sync_copy(ew_t.at[pl.ds(row0, NCH_W)], ewb)
    for b in range(4):
        for hh in range(2):
            pltpu.async_copy(xw.at[srcb.at[b, pl.ds(hh * 64, 64)]],
                             rows_v.at[pl.ds(b * CH + hh * 64, 64)], gsem[b])

    def dinv_body(i, _):
        x = dinv_v[pl.ds(i * 16, 16)] + 1.0
        dinv_v[pl.ds(i * 16, 16)] = _rsqrt_nr(x)
        return 0
    lax.fori_loop(0, NP // 16, dinv_body, 0)

    pltpu.make_async_copy(zbuf, spm_sc.at[pl.ds(base, ROWS_T)], zsem).wait()
    plsc.subcore_barrier()

    def ring_body(i, _):
        for b in range(4):
            g = i * 4 + b
            pb = (b - 1) % 4

            @pl.when(g >= 4)
            def _():
                pltpu.make_async_copy(crow4.at[pl.ds(b * CH, CH)],
                                      spm_sc.at[srcb.at[g]], csem).wait()

            for k in range(CH // 16):
                sl = pl.ds(k * 16, 16)
                sv = srcb[g, sl]
                dv = dstb[g, sl]
                ev = ewb[g, sl]
                mrow4[pl.ds(b * CH + k * 16, 16)] = ev * plsc.load_gather(dinv_v, [sv])
                crow4[pl.ds(b * CH + k * 16, 16)] = ev * plsc.load_gather(dinv_v, [dv])

            @pl.when(jnp.logical_and(g >= 1, g + 3 < NCH_W))
            def _():
                pltpu.make_async_copy(rows_v.at[pl.ds(pb * CH, CH)],
                                      spm_acc.at[dstb.at[g]], ssem[pb]).wait()
                for hh in range(2):
                    pltpu.async_copy(xw.at[srcb.at[g + 3, pl.ds(hh * 64, 64)]],
                                     rows_v.at[pl.ds(pb * CH + hh * 64, 64)], gsem[pb])

            pltpu.make_async_copy(xw.at[srcb.at[g]],
                                  rows_v.at[pl.ds(b * CH, CH)], gsem[b]).wait()
            def scale_body(e, _):
                m = plsc.load_gather(mrow4, [jnp.full((16,), b * CH, jnp.int32) + e])
                for k4 in range(4):
                    sl = pl.ds(k4 * 16, 16)
                    rows_v[b * CH + e, sl] = rows_v[b * CH + e, sl] * m
                return 0
            lax.fori_loop(0, CH, scale_body, 0)

            pltpu.async_copy(rows_v.at[pl.ds(b * CH, CH)],
                             spm_acc.at[dstb.at[g]], ssem[b], add=True)
            pltpu.async_copy(crow4.at[pl.ds(b * CH, CH)], spm_sc.at[srcb.at[g]], csem, add=True)
        return 0
    lax.fori_loop(0, NCH_W // 4, ring_body, 0)

    for b in range(4):
        pltpu.make_async_copy(rows_v.at[pl.ds(b * CH, CH)],
                              spm_acc.at[dstb.at[NCH_W - 4 + b]], ssem[b]).wait()
        pltpu.make_async_copy(crow4.at[pl.ds(b * CH, CH)],
                              spm_sc.at[srcb.at[NCH_W - 4 + b]], csem).wait()
    plsc.subcore_barrier()

    obase = c * NP + base
    pltpu.sync_copy(spm_acc.at[pl.ds(base, ROWS_T)], acc_out.at[pl.ds(obase, ROWS_T)])
    pltpu.sync_copy(spm_sc.at[pl.ds(base, ROWS_T)], c2_out.at[pl.ds(obase, ROWS_T)])


def _mm_body(x_ref, w_ref, o_ref):
    o_ref[...] = jnp.dot(x_ref[...], w_ref[...], preferred_element_type=jnp.float32)


def _epi_body(deg_ref, acc0_ref, acc1_ref, xw_ref, c20_ref, c21_ref,
              b1_ref, wmu_ref, bmu_ref, wc1_ref, bc1_ref, wc2_ref, bc2_ref,
              o_ref, s_acc):
    i = pl.program_id(0)

    @pl.when(i == 0)
    def _():
        s_acc[...] = jnp.zeros_like(s_acc)

    dinv = lax.rsqrt(deg_ref[...] + 1.0)
    xw = xw_ref[...]
    h = jnp.maximum(dinv * (acc0_ref[...] + acc1_ref[...])
                    + dinv * dinv * xw + b1_ref[...], 0.0)
    coeff = dinv * (c20_ref[...] + c21_ref[...]) + dinv * dinv
    rid = i * deg_ref.shape[0] + lax.broadcasted_iota(jnp.int32, coeff.shape, 0)
    coeff = jnp.where(rid < N, coeff, 0.0)
    s_acc[...] += jnp.sum(coeff * h, axis=0, keepdims=True)

    @pl.when(i == pl.num_programs(0) - 1)
    def _():
        zg = jnp.dot(s_acc[...], wmu_ref[...],
                     preferred_element_type=jnp.float32) * (1.0 / N) + bmu_ref[...]
        h2 = jnp.maximum(jnp.dot(zg, wc1_ref[...],
                                 preferred_element_type=jnp.float32) + bc1_ref[...], 0.0)
        logits = jnp.dot(h2, wc2_ref[...],
                         preferred_element_type=jnp.float32) + bc2_ref[...]
        o_ref[...] = 1.0 / (1.0 + jnp.exp(-logits))


@jax.jit
def kernel(x, edge_index, edge_weight, W1, b1, Wmu, bmu, Wc1, bc1, Wc2, bc2):
    xp = jnp.pad(x, ((0, NP - N), (0, 0)))
    src = jnp.pad(edge_index[0], (0, EPAD - E)).reshape(NW * NCH_W, CH)
    dst = jnp.pad(edge_index[1], (0, EPAD - E)).reshape(NW * NCH_W, CH)
    ew = jnp.pad(edge_weight, (0, EPAD - E)).reshape(NW * NCH_W, CH)

    BR = 512
    xw = pl.pallas_call(
        _mm_body,
        grid=(NP // BR,),
        in_specs=[pl.BlockSpec((BR, D_FEAT), lambda i: (i, 0)),
                  pl.BlockSpec((D_FEAT, HIDDEN), lambda i: (0, 0))],
        out_specs=pl.BlockSpec((BR, HIDDEN), lambda i: (i, 0)),
        out_shape=jax.ShapeDtypeStruct((NP, HIDDEN), jnp.float32),
    )(xp, W1)

    mesh = plsc.VectorSubcoreMesh(core_axis_name="c", subcore_axis_name="s")
    acc, c2, deg = pl.kernel(
        _sc_kernel,
        mesh=mesh,
        compiler_params=pltpu.CompilerParams(needs_layout_passes=False,
                                             use_tc_tiling_on_sc=False),
        out_type=[
            jax.ShapeDtypeStruct((2 * NP, HIDDEN), jnp.float32),
            jax.ShapeDtypeStruct((2 * NP,), jnp.float32),
            jax.ShapeDtypeStruct((NP,), jnp.float32),
        ],
        scratch_types=[
            pltpu.VMEM_SHARED((NP, HIDDEN), jnp.float32),
            pltpu.VMEM_SHARED((NP,), jnp.float32),
            pltpu.VMEM((NP,), jnp.float32),
            pltpu.VMEM((ROWS_T,), jnp.float32),
            pltpu.VMEM((NCH_W, CH), jnp.int32),
            pltpu.VMEM((NCH_W, CH), jnp.int32),
            pltpu.VMEM((NCH_W, CH), jnp.float32),
            pltpu.VMEM((4 * CH, HIDDEN), jnp.float32),
            pltpu.VMEM((4 * CH,), jnp.float32),
            pltpu.VMEM((4 * CH,), jnp.float32),
        ] + [pltpu.SemaphoreType.DMA] * 11,
    )(src, dst, ew, xw)

    BN = 512
    deg2 = deg[:, None]
    acc0, acc1 = acc[:NP], acc[NP:]
    c20, c21 = c2[:NP, None], c2[NP:, None]
    grid = (NP // BN,)
    bcast = lambda i: (0, 0)
    out = pl.pallas_call(
        _epi_body,
        grid=grid,
        in_specs=[
            pl.BlockSpec((BN, 1), lambda i: (i, 0)),
            pl.BlockSpec((BN, HIDDEN), lambda i: (i, 0)),
            pl.BlockSpec((BN, HIDDEN), lambda i: (i, 0)),
            pl.BlockSpec((BN, HIDDEN), lambda i: (i, 0)),
            pl.BlockSpec((BN, 1), lambda i: (i, 0)),
            pl.BlockSpec((BN, 1), lambda i: (i, 0)),
            pl.BlockSpec((1, HIDDEN), bcast),
            pl.BlockSpec((HIDDEN, 32), bcast),
            pl.BlockSpec((1, 32), bcast),
            pl.BlockSpec((32, HIDDEN), bcast),
            pl.BlockSpec((1, HIDDEN), bcast),
            pl.BlockSpec((HIDDEN, 1), bcast),
            pl.BlockSpec((1, 1), bcast),
        ],
        out_specs=pl.BlockSpec((1, 1), bcast),
        out_shape=jax.ShapeDtypeStruct((1, 1), jnp.float32),
        scratch_shapes=[pltpu.VMEM((1, HIDDEN), jnp.float32)],
    )(deg2, acc0, acc1, xw, c20, c21,
      b1[None, :], Wmu, bmu[None, :], Wc1, bc1[None, :], Wc2, bc2[None, :])
    return out


# speedup vs baseline: 1.3077x; 1.3077x over previous
import functools

import jax
import jax.numpy as jnp
from jax import lax
from jax.experimental import pallas as pl
from jax.experimental.pallas import tpu as pltpu
from jax.experimental.pallas import tpu_sc as plsc

N = 10000
NP = 10240
E = 320000
D_FEAT = 128
HIDDEN = 64
CH = 128
NW = 32
NCH_W = 80
EPAD = NW * NCH_W * CH
NCH_S = 160
SB = 20
ROWS_T = NP // 16


def _rsqrt_nr(x):
    i = jnp.int32(0x5F3759DF) - (lax.bitcast_convert_type(x, jnp.int32) >> 1)
    r = lax.bitcast_convert_type(i, jnp.float32)
    for _ in range(3):
        r = r * (1.5 - 0.5 * x * r * r)
    return r


def _sc_kernel(src_t, dst_t, ew_t, xw, acc_out, c2_out, deg_out,
               spm_xw, spm_acc, spm_sc, dinv_v, zbuf, srcb, dstb, ewb,
               rows_v, mrow2, crow2,
               gsem0, gsem1, ssem0, ssem1, csem, dsem, zsem):
    c = lax.axis_index("c")
    s = lax.axis_index("s")
    wid = s * 2 + c
    base = s * ROWS_T
    gsem = [gsem0, gsem1]
    ssem = [ssem0, ssem1]

    pltpu.async_copy(xw.at[pl.ds(base, ROWS_T)], spm_xw.at[pl.ds(base, ROWS_T)], zsem)

    def zrow(r, _):
        for k4 in range(4):
            rows_v[r, pl.ds(k4 * 16, 16)] = jnp.zeros((16,), jnp.float32)
        return 0
    lax.fori_loop(0, CH, zrow, 0)

    def zflat(i, _):
        zbuf[pl.ds(i * 16, 16)] = jnp.zeros((16,), jnp.float32)
        return 0
    lax.fori_loop(0, ROWS_T // 16, zflat, 0)

    for k in range(ROWS_T // CH):
        pltpu.async_copy(rows_v.at[pl.ds(0, CH)], spm_acc.at[pl.ds(base + k * CH, CH)], zsem)
    pltpu.async_copy(zbuf, spm_sc.at[pl.ds(base, ROWS_T)], zsem)
    pltpu.make_async_copy(xw.at[pl.ds(base, ROWS_T)], spm_xw.at[pl.ds(base, ROWS_T)], zsem).wait()
    for k in range(ROWS_T // CH):
        pltpu.make_async_copy(rows_v.at[pl.ds(0, CH)], spm_acc.at[pl.ds(base + k * CH, CH)], zsem).wait()
    pltpu.make_async_copy(zbuf, spm_sc.at[pl.ds(base, ROWS_T)], zsem).wait()
    plsc.subcore_barrier()

    for hb in range(NCH_S // SB):
        row0 = s * NCH_S + hb * SB
        pltpu.sync_copy(dst_t.at[pl.ds(row0, SB)], dstb)
        pltpu.sync_copy(ew_t.at[pl.ds(row0, SB)], ewb)

        def deg_body(g, _):
            pltpu.async_copy(ewb.at[g], spm_sc.at[dstb.at[g]], dsem, add=True)
            return 0
        lax.fori_loop(0, SB, deg_body, 0)

        def deg_drain(g, _):
            pltpu.make_async_copy(ewb.at[g], spm_sc.at[dstb.at[g]], dsem).wait()
            return 0
        lax.fori_loop(0, SB, deg_drain, 0)
    plsc.subcore_barrier()

    pltpu.sync_copy(spm_sc, dinv_v)

    @pl.when(c == 0)
    def _():
        pltpu.sync_copy(spm_sc.at[pl.ds(base, ROWS_T)], deg_out.at[pl.ds(base, ROWS_T)])
    plsc.subcore_barrier()
    pltpu.async_copy(zbuf, spm_sc.at[pl.ds(base, ROWS_T)], zsem)

    def dinv_body(i, _):
        x = dinv_v[pl.ds(i * 16, 16)] + 1.0
        dinv_v[pl.ds(i * 16, 16)] = _rsqrt_nr(x)
        return 0
    lax.fori_loop(0, NP // 16, dinv_body, 0)

    pltpu.make_async_copy(zbuf, spm_sc.at[pl.ds(base, ROWS_T)], zsem).wait()
    plsc.subcore_barrier()

    for sb in range(NCH_W // SB):
        row0 = wid * NCH_W + sb * SB
        pltpu.sync_copy(src_t.at[pl.ds(row0, SB)], srcb)
        pltpu.sync_copy(dst_t.at[pl.ds(row0, SB)], dstb)
        pltpu.sync_copy(ew_t.at[pl.ds(row0, SB)], ewb)
        for b in range(2):
            pltpu.async_copy(spm_xw.at[srcb.at[b]], rows_v.at[pl.ds(b * CH, CH)], gsem[b])

        def ring_body(i, _):
            for b in range(2):
                g = i * 2 + b
                pb = 1 - b

                @pl.when(g >= 2)
                def _():
                    pltpu.make_async_copy(crow2.at[pl.ds(b * CH, CH)],
                                          spm_sc.at[srcb.at[g]], csem).wait()

                for k in range(CH // 16):
                    sl = pl.ds(k * 16, 16)
                    sv = srcb[g, sl]
                    dv = dstb[g, sl]
                    ev = ewb[g, sl]
                    mrow2[pl.ds(b * CH + k * 16, 16)] = ev * plsc.load_gather(dinv_v, [sv])
                    crow2[pl.ds(b * CH + k * 16, 16)] = ev * plsc.load_gather(dinv_v, [dv])

                @pl.when(jnp.logical_and(g >= 1, g + 1 < SB))
                def _():
                    pltpu.make_async_copy(rows_v.at[pl.ds(pb * CH, CH)],
                                          spm_acc.at[dstb.at[g]], ssem[pb]).wait()
                    pltpu.async_copy(spm_xw.at[srcb.at[g + 1]],
                                     rows_v.at[pl.ds(pb * CH, CH)], gsem[pb])

                pltpu.make_async_copy(spm_xw.at[srcb.at[g]],
                                      rows_v.at[pl.ds(b * CH, CH)], gsem[b]).wait()

                def scale_body(e, _):
                    m = plsc.load_gather(mrow2, [jnp.full((16,), b * CH, jnp.int32) + e])
                    for k4 in range(4):
                        sl = pl.ds(k4 * 16, 16)
                        rows_v[b * CH + e, sl] = rows_v[b * CH + e, sl] * m
                    return 0
                lax.fori_loop(0, CH, scale_body, 0)

                pltpu.async_copy(rows_v.at[pl.ds(b * CH, CH)],
                                 spm_acc.at[dstb.at[g]], ssem[b], add=True)
                pltpu.async_copy(crow2.at[pl.ds(b * CH, CH)], spm_sc.at[srcb.at[g]], csem, add=True)
            return 0
        lax.fori_loop(0, SB // 2, ring_body, 0)

        for b in range(2):
            pltpu.make_async_copy(rows_v.at[pl.ds(b * CH, CH)],
                                  spm_acc.at[dstb.at[SB - 2 + b]], ssem[b]).wait()
            pltpu.make_async_copy(crow2.at[pl.ds(b * CH, CH)],
                                  spm_sc.at[srcb.at[SB - 2 + b]], csem).wait()
    plsc.subcore_barrier()

    obase = c * NP + base
    pltpu.sync_copy(spm_acc.at[pl.ds(base, ROWS_T)], acc_out.at[pl.ds(obase, ROWS_T)])
    pltpu.sync_copy(spm_sc.at[pl.ds(base, ROWS_T)], c2_out.at[pl.ds(obase, ROWS_T)])


def _mm_body(x_ref, w_ref, o_ref):
    o_ref[...] = jnp.dot(x_ref[...], w_ref[...], preferred_element_type=jnp.float32)


def _epi_body(deg_ref, acc0_ref, acc1_ref, xw_ref, c20_ref, c21_ref,
              b1_ref, wmu_ref, bmu_ref, wc1_ref, bc1_ref, wc2_ref, bc2_ref,
              o_ref, s_acc):
    i = pl.program_id(0)

    @pl.when(i == 0)
    def _():
        s_acc[...] = jnp.zeros_like(s_acc)

    dinv = lax.rsqrt(deg_ref[...] + 1.0)
    xw = xw_ref[...]
    h = jnp.maximum(dinv * (acc0_ref[...] + acc1_ref[...])
                    + dinv * dinv * xw + b1_ref[...], 0.0)
    coeff = dinv * (c20_ref[...] + c21_ref[...]) + dinv * dinv
    rid = i * deg_ref.shape[0] + lax.broadcasted_iota(jnp.int32, coeff.shape, 0)
    coeff = jnp.where(rid < N, coeff, 0.0)
    s_acc[...] += jnp.sum(coeff * h, axis=0, keepdims=True)

    @pl.when(i == pl.num_programs(0) - 1)
    def _():
        zg = jnp.dot(s_acc[...], wmu_ref[...],
                     preferred_element_type=jnp.float32) * (1.0 / N) + bmu_ref[...]
        h2 = jnp.maximum(jnp.dot(zg, wc1_ref[...],
                                 preferred_element_type=jnp.float32) + bc1_ref[...], 0.0)
        logits = jnp.dot(h2, wc2_ref[...],
                         preferred_element_type=jnp.float32) + bc2_ref[...]
        o_ref[...] = 1.0 / (1.0 + jnp.exp(-logits))


@jax.jit
def kernel(x, edge_index, edge_weight, W1, b1, Wmu, bmu, Wc1, bc1, Wc2, bc2):
    xp = jnp.pad(x, ((0, NP - N), (0, 0)))
    src = jnp.pad(edge_index[0], (0, EPAD - E)).reshape(NW * NCH_W, CH)
    dst = jnp.pad(edge_index[1], (0, EPAD - E)).reshape(NW * NCH_W, CH)
    ew = jnp.pad(edge_weight, (0, EPAD - E)).reshape(NW * NCH_W, CH)

    BR = 512
    xw = pl.pallas_call(
        _mm_body,
        grid=(NP // BR,),
        in_specs=[pl.BlockSpec((BR, D_FEAT), lambda i: (i, 0)),
                  pl.BlockSpec((D_FEAT, HIDDEN), lambda i: (0, 0))],
        out_specs=pl.BlockSpec((BR, HIDDEN), lambda i: (i, 0)),
        out_shape=jax.ShapeDtypeStruct((NP, HIDDEN), jnp.float32),
    )(xp, W1)

    mesh = plsc.VectorSubcoreMesh(core_axis_name="c", subcore_axis_name="s")
    acc, c2, deg = pl.kernel(
        _sc_kernel,
        mesh=mesh,
        compiler_params=pltpu.CompilerParams(needs_layout_passes=False,
                                             use_tc_tiling_on_sc=False),
        out_type=[
            jax.ShapeDtypeStruct((2 * NP, HIDDEN), jnp.float32),
            jax.ShapeDtypeStruct((2 * NP,), jnp.float32),
            jax.ShapeDtypeStruct((NP,), jnp.float32),
        ],
        scratch_types=[
            pltpu.VMEM_SHARED((NP, HIDDEN), jnp.float32),
            pltpu.VMEM_SHARED((NP, HIDDEN), jnp.float32),
            pltpu.VMEM_SHARED((NP,), jnp.float32),
            pltpu.VMEM((NP,), jnp.float32),
            pltpu.VMEM((ROWS_T,), jnp.float32),
            pltpu.VMEM((SB, CH), jnp.int32),
            pltpu.VMEM((SB, CH), jnp.int32),
            pltpu.VMEM((SB, CH), jnp.float32),
            pltpu.VMEM((2 * CH, HIDDEN), jnp.float32),
            pltpu.VMEM((2 * CH,), jnp.float32),
            pltpu.VMEM((2 * CH,), jnp.float32),
        ] + [pltpu.SemaphoreType.DMA] * 7,
    )(src, dst, ew, xw)

    BN = 512
    deg2 = deg[:, None]
    acc0, acc1 = acc[:NP], acc[NP:]
    c20, c21 = c2[:NP, None], c2[NP:, None]
    grid = (NP // BN,)
    bcast = lambda i: (0, 0)
    out = pl.pallas_call(
        _epi_body,
        grid=grid,
        in_specs=[
            pl.BlockSpec((BN, 1), lambda i: (i, 0)),
            pl.BlockSpec((BN, HIDDEN), lambda i: (i, 0)),
            pl.BlockSpec((BN, HIDDEN), lambda i: (i, 0)),
            pl.BlockSpec((BN, HIDDEN), lambda i: (i, 0)),
            pl.BlockSpec((BN, 1), lambda i: (i, 0)),
            pl.BlockSpec((BN, 1), lambda i: (i, 0)),
            pl.BlockSpec((1, HIDDEN), bcast),
            pl.BlockSpec((HIDDEN, 32), bcast),
            pl.BlockSpec((1, 32), bcast),
            pl.BlockSpec((32, HIDDEN), bcast),
            pl.BlockSpec((1, HIDDEN), bcast),
            pl.BlockSpec((HIDDEN, 1), bcast),
            pl.BlockSpec((1, 1), bcast),
        ],
        out_specs=pl.BlockSpec((1, 1), bcast),
        out_shape=jax.ShapeDtypeStruct((1, 1), jnp.float32),
        scratch_shapes=[pltpu.VMEM((1, HIDDEN), jnp.float32)],
    )(deg2, acc0, acc1, xw, c20, c21,
      b1[None, :], Wmu, bmu[None, :], Wc1, bc1[None, :], Wc2, bc2[None, :])
    return out


# speedup vs baseline: 1.3326x; 1.0191x over previous
import functools

import jax
import jax.numpy as jnp
from jax import lax
from jax.experimental import pallas as pl
from jax.experimental.pallas import tpu as pltpu
from jax.experimental.pallas import tpu_sc as plsc

N = 10000
NP = 10240
E = 320000
D_FEAT = 128
HIDDEN = 64
CH = 128
NW = 32
NCH_W = 80
EPAD = NW * NCH_W * CH
NCH_S = 160
SB = 20
ROWS_T = NP // 16


def _rsqrt_nr(x):
    i = jnp.int32(0x5F3759DF) - (lax.bitcast_convert_type(x, jnp.int32) >> 1)
    r = lax.bitcast_convert_type(i, jnp.float32)
    for _ in range(3):
        r = r * (1.5 - 0.5 * x * r * r)
    return r


def _sc_kernel(src_t, dst_t, ew_t, xw, acc_out, c2_out, deg_out,
               spm_xw, spm_acc, spm_sc, dinv_v, zbuf, srcb, dstb, ewb,
               rows_v, mrow2, crow2,
               gsem0, gsem1, ssem0, ssem1, csem, dsem, zsem):
    c = lax.axis_index("c")
    s = lax.axis_index("s")
    wid = s * 2 + c
    base = s * ROWS_T
    gsem = [gsem0, gsem1]
    ssem = [ssem0, ssem1]

    pltpu.async_copy(xw.at[pl.ds(base, ROWS_T)], spm_xw.at[pl.ds(base, ROWS_T)], zsem)

    def zrow(r, _):
        for k4 in range(4):
            rows_v[r, pl.ds(k4 * 16, 16)] = jnp.zeros((16,), jnp.float32)
        return 0
    lax.fori_loop(0, CH, zrow, 0)

    def zflat(i, _):
        zbuf[pl.ds(i * 16, 16)] = jnp.zeros((16,), jnp.float32)
        return 0
    lax.fori_loop(0, ROWS_T // 16, zflat, 0)

    for k in range(ROWS_T // CH):
        pltpu.async_copy(rows_v.at[pl.ds(0, CH)], spm_acc.at[pl.ds(base + k * CH, CH)], zsem)
    pltpu.async_copy(zbuf, spm_sc.at[pl.ds(base, ROWS_T)], zsem)
    pltpu.make_async_copy(xw.at[pl.ds(base, ROWS_T)], spm_xw.at[pl.ds(base, ROWS_T)], zsem).wait()
    for k in range(ROWS_T // CH):
        pltpu.make_async_copy(rows_v.at[pl.ds(0, CH)], spm_acc.at[pl.ds(base + k * CH, CH)], zsem).wait()
    pltpu.make_async_copy(zbuf, spm_sc.at[pl.ds(base, ROWS_T)], zsem).wait()
    plsc.subcore_barrier()

    for hb in range(NCH_S // SB):
        row0 = s * NCH_S + hb * SB
        pltpu.sync_copy(dst_t.at[pl.ds(row0, SB)], dstb)
        pltpu.sync_copy(ew_t.at[pl.ds(row0, SB)], ewb)

        def deg_body(g, _):
            pltpu.async_copy(ewb.at[g], spm_sc.at[dstb.at[g]], dsem, add=True)
            return 0
        lax.fori_loop(0, SB, deg_body, 0)

        def deg_drain(g, _):
            pltpu.make_async_copy(ewb.at[g], spm_sc.at[dstb.at[g]], dsem).wait()
            return 0
        lax.fori_loop(0, SB, deg_drain, 0)
    plsc.subcore_barrier()

    pltpu.sync_copy(spm_sc, dinv_v)

    @pl.when(c == 0)
    def _():
        pltpu.sync_copy(spm_sc.at[pl.ds(base, ROWS_T)], deg_out.at[pl.ds(base, ROWS_T)])
    plsc.subcore_barrier()
    pltpu.async_copy(zbuf, spm_sc.at[pl.ds(base, ROWS_T)], zsem)

    def dinv_body(i, _):
        x = dinv_v[pl.ds(i * 16, 16)] + 1.0
        dinv_v[pl.ds(i * 16, 16)] = _rsqrt_nr(x)
        return 0
    lax.fori_loop(0, NP // 16, dinv_body, 0)

    pltpu.make_async_copy(zbuf, spm_sc.at[pl.ds(base, ROWS_T)], zsem).wait()
    plsc.subcore_barrier()

    for sb in range(NCH_W // SB):
        row0 = wid * NCH_W + sb * SB
        pltpu.sync_copy(src_t.at[pl.ds(row0, SB)], srcb)
        pltpu.sync_copy(dst_t.at[pl.ds(row0, SB)], dstb)
        pltpu.sync_copy(ew_t.at[pl.ds(row0, SB)], ewb)
        for b in range(2):
            pltpu.async_copy(spm_xw.at[srcb.at[b]], rows_v.at[pl.ds(b * CH, CH)], gsem[b])

        def ring_body(i, _):
            for b in range(2):
                g = i * 2 + b
                pb = 1 - b

                @pl.when(g >= 2)
                def _():
                    pltpu.make_async_copy(crow2.at[pl.ds(b * CH, CH)],
                                          spm_sc.at[srcb.at[g]], csem).wait()

                for k in range(CH // 16):
                    sl = pl.ds(k * 16, 16)
                    sv = srcb[g, sl]
                    dv = dstb[g, sl]
                    ev = ewb[g, sl]
                    mrow2[pl.ds(b * CH + k * 16, 16)] = ev * plsc.load_gather(dinv_v, [sv])
                    crow2[pl.ds(b * CH + k * 16, 16)] = ev * plsc.load_gather(dinv_v, [dv])

                @pl.when(jnp.logical_and(g >= 1, g + 1 < SB))
                def _():
                    pltpu.make_async_copy(rows_v.at[pl.ds(pb * CH, CH)],
                                          spm_acc.at[dstb.at[g]], ssem[pb]).wait()
                    pltpu.async_copy(spm_xw.at[srcb.at[g + 1]],
                                     rows_v.at[pl.ds(pb * CH, CH)], gsem[pb])

                pltpu.make_async_copy(spm_xw.at[srcb.at[g]],
                                      rows_v.at[pl.ds(b * CH, CH)], gsem[b]).wait()

                def scale_body(e, idxv):
                    for u in range(4):
                        m = plsc.load_gather(mrow2, [idxv + u])
                        for k4 in range(4):
                            sl = pl.ds(k4 * 16, 16)
                            r = b * CH + e * 4 + u
                            rows_v[r, sl] = rows_v[r, sl] * m
                    return idxv + 4
                lax.fori_loop(0, CH // 4, scale_body,
                              jnp.full((16,), b * CH, jnp.int32))

                pltpu.async_copy(rows_v.at[pl.ds(b * CH, CH)],
                                 spm_acc.at[dstb.at[g]], ssem[b], add=True)
                pltpu.async_copy(crow2.at[pl.ds(b * CH, CH)], spm_sc.at[srcb.at[g]], csem, add=True)
            return 0
        lax.fori_loop(0, SB // 2, ring_body, 0)

        for b in range(2):
            pltpu.make_async_copy(rows_v.at[pl.ds(b * CH, CH)],
                                  spm_acc.at[dstb.at[SB - 2 + b]], ssem[b]).wait()
            pltpu.make_async_copy(crow2.at[pl.ds(b * CH, CH)],
                                  spm_sc.at[srcb.at[SB - 2 + b]], csem).wait()
    plsc.subcore_barrier()

    obase = c * NP + base
    pltpu.sync_copy(spm_acc.at[pl.ds(base, ROWS_T)], acc_out.at[pl.ds(obase, ROWS_T)])
    pltpu.sync_copy(spm_sc.at[pl.ds(base, ROWS_T)], c2_out.at[pl.ds(obase, ROWS_T)])


def _mm_body(x_ref, w_ref, o_ref):
    o_ref[...] = jnp.dot(x_ref[...], w_ref[...], preferred_element_type=jnp.float32)


def _epi_body(deg_ref, acc0_ref, acc1_ref, xw_ref, c20_ref, c21_ref,
              b1_ref, wmu_ref, bmu_ref, wc1_ref, bc1_ref, wc2_ref, bc2_ref,
              o_ref, s_acc):
    i = pl.program_id(0)

    @pl.when(i == 0)
    def _():
        s_acc[...] = jnp.zeros_like(s_acc)

    dinv = lax.rsqrt(deg_ref[...] + 1.0)
    xw = xw_ref[...]
    h = jnp.maximum(dinv * (acc0_ref[...] + acc1_ref[...])
                    + dinv * dinv * xw + b1_ref[...], 0.0)
    coeff = dinv * (c20_ref[...] + c21_ref[...]) + dinv * dinv
    rid = i * deg_ref.shape[0] + lax.broadcasted_iota(jnp.int32, coeff.shape, 0)
    coeff = jnp.where(rid < N, coeff, 0.0)
    s_acc[...] += jnp.sum(coeff * h, axis=0, keepdims=True)

    @pl.when(i == pl.num_programs(0) - 1)
    def _():
        zg = jnp.dot(s_acc[...], wmu_ref[...],
                     preferred_element_type=jnp.float32) * (1.0 / N) + bmu_ref[...]
        h2 = jnp.maximum(jnp.dot(zg, wc1_ref[...],
                                 preferred_element_type=jnp.float32) + bc1_ref[...], 0.0)
        logits = jnp.dot(h2, wc2_ref[...],
                         preferred_element_type=jnp.float32) + bc2_ref[...]
        o_ref[...] = 1.0 / (1.0 + jnp.exp(-logits))


@jax.jit
def kernel(x, edge_index, edge_weight, W1, b1, Wmu, bmu, Wc1, bc1, Wc2, bc2):
    xp = jnp.pad(x, ((0, NP - N), (0, 0)))
    src = jnp.pad(edge_index[0], (0, EPAD - E)).reshape(NW * NCH_W, CH)
    dst = jnp.pad(edge_index[1], (0, EPAD - E)).reshape(NW * NCH_W, CH)
    ew = jnp.pad(edge_weight, (0, EPAD - E)).reshape(NW * NCH_W, CH)

    BR = 512
    xw = pl.pallas_call(
        _mm_body,
        grid=(NP // BR,),
        in_specs=[pl.BlockSpec((BR, D_FEAT), lambda i: (i, 0)),
                  pl.BlockSpec((D_FEAT, HIDDEN), lambda i: (0, 0))],
        out_specs=pl.BlockSpec((BR, HIDDEN), lambda i: (i, 0)),
        out_shape=jax.ShapeDtypeStruct((NP, HIDDEN), jnp.float32),
    )(xp, W1)

    mesh = plsc.VectorSubcoreMesh(core_axis_name="c", subcore_axis_name="s")
    acc, c2, deg = pl.kernel(
        _sc_kernel,
        mesh=mesh,
        compiler_params=pltpu.CompilerParams(needs_layout_passes=False,
                                             use_tc_tiling_on_sc=False),
        out_type=[
            jax.ShapeDtypeStruct((2 * NP, HIDDEN), jnp.float32),
            jax.ShapeDtypeStruct((2 * NP,), jnp.float32),
            jax.ShapeDtypeStruct((NP,), jnp.float32),
        ],
        scratch_types=[
            pltpu.VMEM_SHARED((NP, HIDDEN), jnp.float32),
            pltpu.VMEM_SHARED((NP, HIDDEN), jnp.float32),
            pltpu.VMEM_SHARED((NP,), jnp.float32),
            pltpu.VMEM((NP,), jnp.float32),
            pltpu.VMEM((ROWS_T,), jnp.float32),
            pltpu.VMEM((SB, CH), jnp.int32),
            pltpu.VMEM((SB, CH), jnp.int32),
            pltpu.VMEM((SB, CH), jnp.float32),
            pltpu.VMEM((2 * CH, HIDDEN), jnp.float32),
            pltpu.VMEM((2 * CH,), jnp.float32),
            pltpu.VMEM((2 * CH,), jnp.float32),
        ] + [pltpu.SemaphoreType.DMA] * 7,
    )(src, dst, ew, xw)

    BN = 512
    deg2 = deg[:, None]
    acc0, acc1 = acc[:NP], acc[NP:]
    c20, c21 = c2[:NP, None], c2[NP:, None]
    grid = (NP // BN,)
    bcast = lambda i: (0, 0)
    out = pl.pallas_call(
        _epi_body,
        grid=grid,
        in_specs=[
            pl.BlockSpec((BN, 1), lambda i: (i, 0)),
            pl.BlockSpec((BN, HIDDEN), lambda i: (i, 0)),
            pl.BlockSpec((BN, HIDDEN), lambda i: (i, 0)),
            pl.BlockSpec((BN, HIDDEN), lambda i: (i, 0)),
            pl.BlockSpec((BN, 1), lambda i: (i, 0)),
            pl.BlockSpec((BN, 1), lambda i: (i, 0)),
            pl.BlockSpec((1, HIDDEN), bcast),
            pl.BlockSpec((HIDDEN, 32), bcast),
            pl.BlockSpec((1, 32), bcast),
            pl.BlockSpec((32, HIDDEN), bcast),
            pl.BlockSpec((1, HIDDEN), bcast),
            pl.BlockSpec((HIDDEN, 1), bcast),
            pl.BlockSpec((1, 1), bcast),
        ],
        out_specs=pl.BlockSpec((1, 1), bcast),
        out_shape=jax.ShapeDtypeStruct((1, 1), jnp.float32),
        scratch_shapes=[pltpu.VMEM((1, HIDDEN), jnp.float32)],
    )(deg2, acc0, acc1, xw, c20, c21,
      b1[None, :], Wmu, bmu[None, :], Wc1, bc1[None, :], Wc2, bc2[None, :])
    return out
